# baseline (speedup 1.0000x reference)
import functools

import numpy as np

import jax
import jax.numpy as jnp
from jax import lax
from jax.experimental import pallas as pl
from jax.experimental.pallas import tpu as pltpu
from jax.experimental.pallas import tpu_sc as plsc

_NROWS = 128
_NCOLS = 8192
_NZEROS = _NCOLS - 4096
_SCALE = 0.8 / (2.0 / 3.0)

_NCHUNK = _NCOLS // 16
_MINT32 = np.int32(-(2**31))


def _sc_body(z_hbm, out_hbm, io_v, key_v, buf_v, hist_v, csum_v):
    nc = 2
    wid = lax.axis_index("s") * nc + lax.axis_index("c")
    rows_per_w = _NROWS // 32

    zero16 = jnp.zeros((16,), jnp.int32)
    ones16 = jnp.ones((16,), jnp.int32)
    iota16 = lax.iota(jnp.int32, 16)

    @plsc.parallel_loop(0, 128, unroll=8)
    def _zero_hist(j):
        hist_v[pl.ds(j * 16, 16)] = zero16

    def _scan_bins(nbins, kt):
        def grp(g, carry):
            found, bsel, before_c, run = carry
            sv = zero16
            for l in range(16):
                j = g * 16 + l
                h = hist_v[pl.ds(j * 16, 16)]
                cs = plsc.cumsum(h)
                csum_v[pl.ds(j * 16, 16)] = cs
                hist_v[pl.ds(j * 16, 16)] = zero16
                sv = jnp.where(iota16 == l, jnp.sum(h), sv)
            cg = plsc.cumsum(sv)
            tot = run + cg
            crossed = tot >= kt
            ci = jnp.where(crossed, ones16, zero16)
            hit = jnp.sum(ci) > 0
            first = crossed & (plsc.cumsum(ci) == 1)
            jsel = g * 16 + jnp.sum(jnp.where(first, iota16, zero16))
            rbc = run + jnp.sum(jnp.where(first, cg - sv, zero16))
            take = hit & (found == 0)
            bsel = jnp.where(take, jsel, bsel)
            before_c = jnp.where(take, rbc, before_c)
            found = found | jnp.where(hit, jnp.int32(1), jnp.int32(0))
            run = run + jnp.sum(sv)
            return found, bsel, before_c, run

        init = (jnp.int32(0), jnp.int32(0), jnp.int32(0), jnp.int32(0))
        _, bsel, before_c, _ = plsc.parallel_loop(
            0, nbins // 256, carry=init)(grp)

        c = csum_v[pl.ds(bsel * 16, 16)]
        tot = before_c + c
        crossed = tot >= kt
        ci = jnp.where(crossed, ones16, zero16)
        first = crossed & (plsc.cumsum(ci) == 1)
        lane = jnp.sum(jnp.where(first, iota16, zero16))
        b = bsel * 16 + lane
        excl = jnp.sum(jnp.where(iota16 == lane - 1, c, zero16))
        incl = jnp.sum(jnp.where(iota16 == lane, c, zero16))
        return b, before_c + excl, incl - excl

    def _row_body(r, c):
        row = wid * rows_per_w + r
        pltpu.sync_copy(z_hbm.at[row], io_v)

        k = jnp.int32(_NZEROS)

        @plsc.parallel_loop(0, _NCHUNK, unroll=8)
        def h1(i):
            zc = io_v[pl.ds(i * 16, 16)]
            y = lax.bitcast_convert_type(zc, jnp.int32)
            v = y ^ ((y >> 31) | _MINT32)
            u = lax.bitcast_convert_type(v, jnp.uint32)
            key_v[pl.ds(i * 16, 16)] = u
            b = plsc.bitcast(u >> jnp.uint32(21), jnp.int32)
            plsc.addupdate_scatter(hist_v, [b], ones16)
        b1, before1, cnt1 = _scan_bins(2048, k)
        b1u = b1.astype(jnp.uint32)

        def cpk(i, run):
            u = key_v[pl.ds(i * 16, 16)]
            match = (u >> jnp.uint32(21)) == b1u
            plsc.store_compressed(buf_v.at[pl.ds(run, 16)], u, mask=match)
            return run + jnp.sum(jnp.where(match, ones16, zero16))

        plsc.parallel_loop(0, _NCHUNK, unroll=8, carry=jnp.int32(0))(cpk)

        j = k - before1
        m = (cnt1 + 15) >> 4
        mask21 = jnp.uint32(0x1FFFFF)
        p = jnp.int32(0)
        cb = jnp.int32(0)
        for bit in range(20, -1, -1):
            p2 = p << 1
            p2u = p2.astype(jnp.uint32)
            sh = jnp.uint32(bit)

            def cchunk(t, c, _p2u=p2u, _sh=sh):
                u = buf_v[pl.ds(t * 16, 16)]
                w = (u & mask21) >> _sh
                valid = (t * 16 + iota16) < cnt1
                hitv = (w == _p2u) & valid
                return c + jnp.sum(jnp.where(hitv, ones16, zero16))

            cnt = lax.fori_loop(0, m, cchunk, jnp.int32(0))
            take0 = (cb + cnt) >= j
            p = jnp.where(take0, p2, p2 + 1)
            cb = jnp.where(take0, cb, cb + cnt)

        t_u = (b1u << jnp.uint32(21)) | p.astype(jnp.uint32)
        need = j - cb

        def outp(i, run_eq):
            u = key_v[pl.ds(i * 16, 16)]
            vi = plsc.bitcast(u, jnp.int32)
            y = jnp.where(vi < 0, vi ^ _MINT32, ~vi)
            zc = lax.bitcast_convert_type(y, jnp.float32)
            lt = u < t_u
            eq = u == t_u
            eqi = jnp.where(eq, ones16, zero16)
            rank = run_eq + (plsc.cumsum(eqi) - eqi)
            zeroed = lt | (eq & (rank < need))
            s = 1.0 / (1.0 + jnp.exp(zc * jnp.float32(-_SCALE)))
            io_v[pl.ds(i * 16, 16)] = jnp.where(zeroed, jnp.float32(0.0), s)
            return run_eq + plsc.all_reduce_population_count(eq)

        plsc.parallel_loop(0, _NCHUNK, unroll=8, carry=zero16)(outp)

        pltpu.sync_copy(io_v, out_hbm.at[row])
        return c

    lax.fori_loop(0, rows_per_w, _row_body, 0)


@jax.jit
def kernel(z_loga):
    mesh = plsc.VectorSubcoreMesh(core_axis_name="c", subcore_axis_name="s")
    f = functools.partial(
        pl.kernel,
        mesh=mesh,
        out_type=jax.ShapeDtypeStruct((_NROWS, _NCOLS), jnp.float32),
        scratch_types=[
            pltpu.VMEM((_NCOLS,), jnp.float32),
            pltpu.VMEM((_NCOLS,), jnp.uint32),
            pltpu.VMEM((_NCOLS + 16,), jnp.uint32),
            pltpu.VMEM((2048,), jnp.int32),
            pltpu.VMEM((2048,), jnp.int32),
        ],
        compiler_params=pltpu.CompilerParams(needs_layout_passes=False),
    )(_sc_body)
    return f(z_loga)


# speedup vs baseline: 1.1529x; 1.1529x over previous
import functools

import numpy as np

import jax
import jax.numpy as jnp
from jax import lax
from jax.experimental import pallas as pl
from jax.experimental.pallas import tpu as pltpu
from jax.experimental.pallas import tpu_sc as plsc

_NROWS = 128
_NCOLS = 8192
_NZEROS = _NCOLS - 4096
_SCALE = 0.8 / (2.0 / 3.0)

_NCHUNK = _NCOLS // 16
_MINT32 = np.int32(-(2**31))


def _sc_body(z_hbm, out_hbm, io_v, key_v, hist_v, csum_v):
    nc = 2
    wid = lax.axis_index("s") * nc + lax.axis_index("c")
    rows_per_w = _NROWS // 32

    zero16 = jnp.zeros((16,), jnp.int32)
    ones16 = jnp.ones((16,), jnp.int32)
    iota16 = lax.iota(jnp.int32, 16)

    @plsc.parallel_loop(0, 128, unroll=8)
    def _zero_hist(j):
        hist_v[pl.ds(j * 16, 16)] = zero16

    def _scan_bins(nbins, kt):
        def grp(g, carry):
            found, bsel, before_c, run = carry
            sv = zero16
            for l in range(16):
                j = g * 16 + l
                h = hist_v[pl.ds(j * 16, 16)]
                cs = plsc.cumsum(h)
                csum_v[pl.ds(j * 16, 16)] = cs
                hist_v[pl.ds(j * 16, 16)] = zero16
                sv = jnp.where(iota16 == l, jnp.sum(h), sv)
            cg = plsc.cumsum(sv)
            tot = run + cg
            crossed = tot >= kt
            ci = jnp.where(crossed, ones16, zero16)
            hit = jnp.sum(ci) > 0
            first = crossed & (plsc.cumsum(ci) == 1)
            jsel = g * 16 + jnp.sum(jnp.where(first, iota16, zero16))
            rbc = run + jnp.sum(jnp.where(first, cg - sv, zero16))
            take = hit & (found == 0)
            bsel = jnp.where(take, jsel, bsel)
            before_c = jnp.where(take, rbc, before_c)
            found = found | jnp.where(hit, jnp.int32(1), jnp.int32(0))
            run = run + jnp.sum(sv)
            return found, bsel, before_c, run

        init = (jnp.int32(0), jnp.int32(0), jnp.int32(0), jnp.int32(0))
        _, bsel, before_c, _ = plsc.parallel_loop(
            0, nbins // 256, carry=init)(grp)

        c = csum_v[pl.ds(bsel * 16, 16)]
        tot = before_c + c
        crossed = tot >= kt
        ci = jnp.where(crossed, ones16, zero16)
        first = crossed & (plsc.cumsum(ci) == 1)
        lane = jnp.sum(jnp.where(first, iota16, zero16))
        b = bsel * 16 + lane
        excl = jnp.sum(jnp.where(iota16 == lane - 1, c, zero16))
        incl = jnp.sum(jnp.where(iota16 == lane, c, zero16))
        return b, before_c + excl, incl - excl

    def _row_body(r, c):
        row = wid * rows_per_w + r
        pltpu.sync_copy(z_hbm.at[row], io_v)

        k = jnp.int32(_NZEROS)

        @plsc.parallel_loop(0, _NCHUNK, unroll=8)
        def h1(i):
            zc = io_v[pl.ds(i * 16, 16)]
            y = lax.bitcast_convert_type(zc, jnp.int32)
            v = y ^ ((y >> 31) | _MINT32)
            u = lax.bitcast_convert_type(v, jnp.uint32)
            key_v[pl.ds(i * 16, 16)] = u
            b = plsc.bitcast(u >> jnp.uint32(21), jnp.int32)
            plsc.addupdate_scatter(hist_v, [b], ones16)
        b1, before1, _ = _scan_bins(2048, k)
        b1u = b1.astype(jnp.uint32)

        @plsc.parallel_loop(0, _NCHUNK, unroll=8)
        def h2(i):
            u = key_v[pl.ds(i * 16, 16)]
            match = (u >> jnp.uint32(21)) == b1u
            b = plsc.bitcast(
                (u >> jnp.uint32(10)) & jnp.uint32(0x7FF), jnp.int32)
            plsc.addupdate_scatter(hist_v, [b], ones16, mask=match)
        b2, before2, _ = _scan_bins(2048, k - before1)
        pref2 = (b1u << jnp.uint32(11)) | b2.astype(jnp.uint32)

        @plsc.parallel_loop(0, _NCHUNK, unroll=8)
        def h3(i):
            u = key_v[pl.ds(i * 16, 16)]
            match = (u >> jnp.uint32(10)) == pref2
            b = plsc.bitcast(u & jnp.uint32(0x3FF), jnp.int32)
            plsc.addupdate_scatter(hist_v, [b], ones16, mask=match)
        b3, before3, cnt_eq = _scan_bins(1024, k - before1 - before2)

        t_u = (pref2 << jnp.uint32(10)) | b3.astype(jnp.uint32)
        need = k - before1 - before2 - before3

        @plsc.parallel_loop(0, _NCHUNK, unroll=8)
        def outp(i):
            u = key_v[pl.ds(i * 16, 16)]
            vi = plsc.bitcast(u, jnp.int32)
            y = vi ^ ((~vi >> 31) | _MINT32)
            zc = lax.bitcast_convert_type(y, jnp.float32)
            zeroed = u <= t_u
            s = 1.0 / (1.0 + jnp.exp(zc * jnp.float32(-_SCALE)))
            io_v[pl.ds(i * 16, 16)] = jnp.where(zeroed, jnp.float32(0.0), s)

        tvec = jnp.zeros((16,), jnp.uint32) | t_u
        tvi = plsc.bitcast(tvec, jnp.int32)
        ty = tvi ^ ((~tvi >> 31) | _MINT32)
        tz = lax.bitcast_convert_type(ty, jnp.float32)
        s_t = 1.0 / (1.0 + jnp.exp(tz * jnp.float32(-_SCALE)))
        fix_m = jnp.where(need < cnt_eq, jnp.int32(_NCHUNK), jnp.int32(0))

        def fixup(t, run_eq):
            u = key_v[pl.ds(t * 16, 16)]
            eq = u == t_u
            eqi = jnp.where(eq, ones16, zero16)
            rank = run_eq + (plsc.cumsum(eqi) - eqi)
            unzero = eq & (rank >= need)
            plsc.store_scatter(io_v, [t * 16 + iota16], s_t, mask=unzero)
            return run_eq + plsc.all_reduce_population_count(eq)

        lax.fori_loop(0, fix_m, fixup, zero16)

        pltpu.sync_copy(io_v, out_hbm.at[row])
        return c

    lax.fori_loop(0, rows_per_w, _row_body, 0)


@jax.jit
def kernel(z_loga):
    mesh = plsc.VectorSubcoreMesh(core_axis_name="c", subcore_axis_name="s")
    f = functools.partial(
        pl.kernel,
        mesh=mesh,
        out_type=jax.ShapeDtypeStruct((_NROWS, _NCOLS), jnp.float32),
        scratch_types=[
            pltpu.VMEM((_NCOLS,), jnp.float32),
            pltpu.VMEM((_NCOLS,), jnp.uint32),
            pltpu.VMEM((2048,), jnp.int32),
            pltpu.VMEM((2048,), jnp.int32),
        ],
        compiler_params=pltpu.CompilerParams(needs_layout_passes=False),
    )(_sc_body)
    return f(z_loga)
